# baseline (speedup 1.0000x reference)
import functools

import jax
import jax.numpy as jnp
from jax import lax
from jax.experimental import pallas as pl
from jax.experimental.pallas import tpu as pltpu
from jax.experimental.pallas import tpu_sc as plsc

VOCAB_SIZE = 1000000
EMB = 64
BATCH = 16384
C = 5

NUM_CORES = 2
NUM_SUBCORES = 16
NW = NUM_CORES * NUM_SUBCORES
B_PER_W = BATCH // NW
CB = 128
N_CHUNKS = B_PER_W // CB


SB_STRIDE = 17
N_OUT_GROUPS = CB * C // 16


def _body(tgt_hbm, ctx_hbm, ttab_hbm, ctab_hbm, out_hbm,
          idx_v, cidx_v, w_rows, c_rows, sbuf, out_v, sem):
    wid = lax.axis_index("s") * NUM_CORES + lax.axis_index("c")
    base = wid * B_PER_W
    iota = lax.iota(jnp.int32, 16)
    iota_sb = iota * SB_STRIDE

    for k in range(N_CHUNKS):
        start = base + k * CB
        pltpu.sync_copy(tgt_hbm.at[pl.ds(start, CB)], idx_v)
        pltpu.sync_copy(ctx_hbm.at[pl.ds(start * C, CB * C)], cidx_v)
        g1 = pltpu.async_copy(ttab_hbm.at[idx_v], w_rows, sem)
        g2 = pltpu.async_copy(ctab_hbm.at[cidx_v], c_rows, sem)
        g1.wait()
        g2.wait()

        def bbody(b, carry):
            w = [w_rows[b, pl.ds(16 * q, 16)] for q in range(4)]
            for c in range(C):
                cc = [c_rows[b * C + c, pl.ds(16 * q, 16)] for q in range(4)]
                s = (w[0] * cc[0] + w[1] * cc[1]) + (w[2] * cc[2] + w[3] * cc[3])
                sbuf[pl.ds((b * C + c) * SB_STRIDE, 16)] = s
            return carry

        lax.fori_loop(0, CB, bbody, 0)

        def gbody(og, carry):
            gb = og * (16 * SB_STRIDE)
            acc = plsc.load_gather(sbuf, [iota_sb + gb])
            for j in range(1, 16):
                acc = acc + plsc.load_gather(sbuf, [iota_sb + (gb + j)])
            out_v[pl.ds(og * 16, 16)] = acc
            return carry

        lax.fori_loop(0, N_OUT_GROUPS, gbody, 0)
        pltpu.sync_copy(out_v, out_hbm.at[pl.ds(start * C, CB * C)])


def kernel(target, context, target_table, context_table):
    tgt = target.reshape(BATCH).astype(jnp.int32)
    ctx = context.reshape(BATCH * C).astype(jnp.int32)

    mesh = plsc.VectorSubcoreMesh(core_axis_name="c", subcore_axis_name="s")
    run = functools.partial(
        pl.kernel,
        mesh=mesh,
        compiler_params=pltpu.CompilerParams(
            needs_layout_passes=False, use_tc_tiling_on_sc=False
        ),
        out_type=jax.ShapeDtypeStruct((BATCH * C,), jnp.float32),
        scratch_types=[
            pltpu.VMEM((CB,), jnp.int32),
            pltpu.VMEM((CB * C,), jnp.int32),
            pltpu.VMEM((CB, EMB), jnp.float32),
            pltpu.VMEM((CB * C, EMB), jnp.float32),
            pltpu.VMEM((CB * C * SB_STRIDE,), jnp.float32),
            pltpu.VMEM((CB * C,), jnp.float32),
            pltpu.SemaphoreType.DMA,
        ],
    )(_body)
    out = run(tgt, ctx, target_table, context_table)
    return out.reshape(BATCH, C)
